# baseline (speedup 1.0000x reference)
import functools

import jax
import jax.numpy as jnp
from jax import lax
from jax.experimental import pallas as pl
from jax.experimental.pallas import tpu as pltpu
from jax.experimental.pallas import tpu_sc as plsc

B = 16384
D = 64
T = 20
NC, NS = 2, 16
NW = NC * NS
BC = B // NW
CP = 128
NCP = BC // CP
PERM32 = list(range(0, 32, 2)) + list(range(1, 32, 2))
CG = 128
NCG = BC // CG
LANES = 16
NG = BC // LANES

_MESH = dict(core_axis_name="c", subcore_axis_name="s",
             num_cores=NC, num_subcores=NS)
_PARAMS = dict(use_tc_tiling_on_sc=False, needs_layout_passes=False)


def _wid_base():
    wid = lax.axis_index("s") * NC + lax.axis_index("c")
    return wid * BC


def _sc_pools_body(tag_t, likedT, dislikedT, allergyT, tagsT,
                   out1, outc,
                   idx_v, rows_v, pout_v, misc_v, sem_g):
    base = _wid_base()
    f32 = jnp.float32
    i32 = jnp.int32

    def pool(pi, idxT_hbm):
        def load_and_fire(c):
            p = lax.rem(c, 2)
            off = pl.multiple_of(base + c * CP, CP)
            pltpu.sync_copy(idxT_hbm.at[:, pl.ds(off, CP)], idx_v.at[p])

            def fire(k, carry):
                pltpu.make_async_copy(tag_t.at[idx_v.at[p, k]],
                                      rows_v.at[p, k], sem_g).start()
                return carry
            lax.fori_loop(0, T, fire, 0)

        def reduce_and_out(c):
            p = lax.rem(c, 2)
            off = pl.multiple_of(base + c * CP, CP)

            def drain(k, carry):
                pltpu.make_async_copy(tag_t.at[idx_v.at[p, 0]],
                                      rows_v.at[p, 0], sem_g).wait()
                return carry
            lax.fori_loop(0, T, drain, 0)

            def red(i, carry):
                s0 = jnp.zeros((LANES,), f32)
                s1 = jnp.zeros((LANES,), f32)
                for k in range(T):
                    x = rows_v[p, k, i, :]
                    a, b = plsc.unpack(x, format=plsc.PackFormat.INTERLEAVED)
                    s0 = s0 + a
                    s1 = s1 + b
                pout_v[i, pl.ds(0, LANES)] = s0
                pout_v[i, pl.ds(LANES, LANES)] = s1
                return carry
            lax.fori_loop(0, CP, red, 0)
            pltpu.sync_copy(pout_v,
                            out1.at[pl.ds(off, CP), pl.ds(32 * pi, 32)])

            colv = jnp.full((LANES,), pi, i32)

            def cnt(g, carry):
                sl = pl.ds(g * LANES, LANES)
                s = jnp.zeros((LANES,), f32)
                for k in range(T):
                    s = s + (idx_v[p, k, sl] != 0).astype(f32)
                rowv = lax.iota(i32, LANES) + (c * CP + g * LANES)
                plsc.store_scatter(misc_v, [rowv, colv], s)
                return carry
            lax.fori_loop(0, CP // LANES, cnt, 0)

        load_and_fire(0)
        load_and_fire(1)

        def body(c, carry):
            reduce_and_out(c)
            @pl.when(c + 2 < NCP)
            def _():
                load_and_fire(c + 2)
            return carry
        lax.fori_loop(0, NCP, body, 0)

    pool(0, likedT)
    pool(1, dislikedT)
    pool(2, allergyT)
    pool(3, tagsT)

    pltpu.sync_copy(misc_v, outc.at[pl.ds(base, BC), pl.ds(0, 16)])


def _sc_plain_body(user_t, dish_t, store_t, cat_t,
                   uid, did, sid, cid,
                   age, utod, price, rating, itod, gid, udy, idy,
                   out2, out3,
                   misc_v, tmpf_v, tmpi_v, gidx_v, g64_v, g32_v, g16_v,
                   s64_v, s32_v, sem_p):
    base = _wid_base()
    f32 = jnp.float32
    i32 = jnp.int32

    def pack_scalar(col, src_hbm, is_int):
        tmp = tmpi_v if is_int else tmpf_v
        pltpu.sync_copy(src_hbm.at[pl.ds(base, BC)], tmp)
        colv = jnp.full((LANES,), col, i32)

        def body(g, carry):
            vals = tmp[pl.ds(g * LANES, LANES)]
            if is_int:
                vals = vals.astype(f32)
            rowv = lax.iota(i32, LANES) + g * LANES
            plsc.store_scatter(misc_v, [rowv, colv], vals)
            return carry
        lax.fori_loop(0, NG, body, 0)

    for col, (src, is_int) in enumerate([
            (age, False), (utod, False), (price, False), (rating, False),
            (itod, False), (gid, True), (udy, True), (idy, True)]):
        pack_scalar(col, src, is_int)

    def plain(table, ids_hbm, out_hbm, col0, width, buf, stage_v):
        bf16 = stage_v is not None

        def stage(c):
            p = lax.rem(c, 2)
            off = pl.multiple_of(base + c * CG, CG)
            pltpu.sync_copy(ids_hbm.at[pl.ds(off, CG)], gidx_v.at[p])
            pltpu.make_async_copy(table.at[gidx_v.at[p]], buf.at[p],
                                  sem_p).start()

        def drain(c):
            p = lax.rem(c, 2)
            off = pl.multiple_of(base + c * CG, CG)
            pltpu.make_async_copy(table.at[gidx_v.at[p]], buf.at[p],
                                  sem_p).wait()
            if bf16:
                def widen(i, carry):
                    for g in range(width // 32):
                        x = buf[p, i, pl.ds(g * 32, 32)]
                        a, b = plsc.unpack(
                            x, format=plsc.PackFormat.INTERLEAVED)
                        stage_v[i, pl.ds(g * 32, LANES)] = a
                        stage_v[i, pl.ds(g * 32 + LANES, LANES)] = b
                    return carry
                lax.fori_loop(0, CG, widen, 0)
                src = stage_v
            else:
                src = buf.at[p]
            pltpu.sync_copy(src,
                            out_hbm.at[pl.ds(off, CG), pl.ds(col0, width)])

        stage(0)

        def body(c, carry):
            @pl.when(c + 1 < NCG)
            def _():
                stage(c + 1)
            drain(c)
            return carry
        lax.fori_loop(0, NCG, body, 0)

    plain(user_t, uid, out2, 0, 64, g64_v, s64_v)
    plain(dish_t, did, out2, 64, 64, g64_v, s64_v)
    plain(store_t, sid, out3, 0, 32, g32_v, s32_v)
    plain(cat_t, cid, out3, 32, 16, g16_v, None)

    pltpu.sync_copy(misc_v, out3.at[pl.ds(base, BC), pl.ds(48, 16)])


@functools.cache
def _sc_pools_call():
    return pl.kernel(
        _sc_pools_body,
        out_type=(
            jax.ShapeDtypeStruct((B, 128), jnp.float32),
            jax.ShapeDtypeStruct((B, 128), jnp.float32),
        ),
        mesh=plsc.VectorSubcoreMesh(**_MESH),
        scratch_types=[
            pltpu.VMEM((2, T, CP), jnp.int32),
            pltpu.VMEM((2, T, CP, 32), jnp.bfloat16),
            pltpu.VMEM((CP, 32), jnp.float32),
            pltpu.VMEM((BC, 16), jnp.float32),
            pltpu.SemaphoreType.DMA,
        ],
        compiler_params=pltpu.CompilerParams(**_PARAMS),
    )


@functools.cache
def _sc_plain_call():
    return pl.kernel(
        _sc_plain_body,
        out_type=(
            jax.ShapeDtypeStruct((B, 128), jnp.float32),
            jax.ShapeDtypeStruct((B, 128), jnp.float32),
        ),
        mesh=plsc.VectorSubcoreMesh(**_MESH),
        scratch_types=[
            pltpu.VMEM((BC, 16), jnp.float32),
            pltpu.VMEM((BC,), jnp.float32),
            pltpu.VMEM((BC,), jnp.int32),
            pltpu.VMEM((2, CG), jnp.int32),
            pltpu.VMEM((2, CG, 64), jnp.bfloat16),
            pltpu.VMEM((2, CG, 32), jnp.bfloat16),
            pltpu.VMEM((2, CG, 16), jnp.float32),
            pltpu.VMEM((CG, 64), jnp.float32),
            pltpu.VMEM((CG, 32), jnp.float32),
            pltpu.SemaphoreType.DMA,
        ],
        compiler_params=pltpu.CompilerParams(**_PARAMS),
    )


BN = 2048


def _tc_body(o1, oc, o2, o3,
             Wu, bu, Wi, bi, gemb, uday, iday,
             ageW, ageb, utW, utb, prW, prb, rtW, rtb, itW, itb, row0,
             uo, io, doto):
    f32 = jnp.float32
    dot = functools.partial(lax.dot, preferred_element_type=f32)
    r0 = row0[...]
    o1_ = o1[...]
    oc_ = oc[...]
    o2_ = o2[...]
    o3_ = o3[...]

    def pool(acc, cnt):
        return (acc - (float(T) - cnt) * r0) / (cnt + 1e-8)

    liked = pool(o1_[:, 0:32], oc_[:, 0:1])
    disl = pool(o1_[:, 32:64], oc_[:, 1:2])
    alle = pool(o1_[:, 64:96], oc_[:, 2:3])
    tagv = pool(o1_[:, 96:128], oc_[:, 3:4])
    age = o3_[:, 48:49]
    utod = o3_[:, 49:50]
    price = o3_[:, 50:51]
    rating = o3_[:, 51:52]
    itod = o3_[:, 52:53]

    def onehot(col, n):
        ci = col.astype(jnp.int32)
        return (lax.broadcasted_iota(jnp.int32, (BN, n), 1) == ci).astype(f32)

    Wu_ = Wu[...]
    u = dot(o2_[:, 0:64], Wu_[0:64])
    u += dot(liked, Wu_[112:144])
    u += dot(disl, Wu_[144:176])
    u += dot(alle, Wu_[176:208])
    u += age * dot(ageW[...], Wu_[64:80])
    u += utod * dot(utW[...], Wu_[96:104])
    u += dot(onehot(o3_[:, 53:54], 3), dot(gemb[...], Wu_[80:96]))
    u += dot(onehot(o3_[:, 54:55], 7), dot(uday[...], Wu_[104:112]))
    u += bu[...] + dot(ageb[...], Wu_[64:80]) + dot(utb[...], Wu_[96:104])
    nu = jnp.sqrt(jnp.sum(u * u, axis=1, keepdims=True))
    un = u / jnp.maximum(nu, 1e-12)

    Wi_ = Wi[...]
    iv = dot(o2_[:, 64:128], Wi_[0:64])
    iv += dot(o3_[:, 0:32], Wi_[64:96])
    iv += dot(tagv, Wi_[96:128])
    iv += dot(o3_[:, 32:48], Wi_[128:144])
    iv += price * dot(prW[...], Wi_[144:160])
    iv += rating * dot(rtW[...], Wi_[160:168])
    iv += itod * dot(itW[...], Wi_[168:176])
    iv += dot(onehot(o3_[:, 55:56], 7), dot(iday[...], Wi_[176:184]))
    iv += (bi[...] + dot(prb[...], Wi_[144:160]) + dot(rtb[...], Wi_[160:168])
           + dot(itb[...], Wi_[168:176]))
    ni = jnp.sqrt(jnp.sum(iv * iv, axis=1, keepdims=True))
    ivn = iv / jnp.maximum(ni, 1e-12)

    uo[...] = un
    io[...] = ivn
    doto[...] = jnp.sum(un * ivn, axis=1, keepdims=True)


def _row_spec(k):
    return pl.BlockSpec((BN, k), lambda i: (i, 0))


def _full_spec(shape):
    return pl.BlockSpec(shape, lambda i: (0,) * len(shape))


def kernel(user_user_id, user_age, user_gender, user_time_of_day,
           user_day_of_week, user_liked_tags, user_disliked_tags,
           user_allergy_tags, item_dish_id, item_store_id, item_category,
           item_tags, item_price, item_rating, item_time_of_day,
           item_day_of_week, user_embedding, user_age_W, user_age_b,
           user_gender_emb, user_time_W, user_time_b, user_day_emb,
           dish_embedding, store_embedding, category_embedding,
           dish_price_W, dish_price_b, dish_rating_W, dish_rating_b,
           dish_time_W, dish_time_b, dish_day_emb, tag_embedding,
           user_proj_W, user_proj_b, item_proj_W, item_proj_b):
    i32 = jnp.int32
    f32 = jnp.float32

    o1, oc = _sc_pools_call()(
        tag_embedding.astype(jnp.bfloat16),
        user_liked_tags.astype(i32).T, user_disliked_tags.astype(i32).T,
        user_allergy_tags.astype(i32).T, item_tags.astype(i32).T)

    bf16 = jnp.bfloat16
    o2, o3 = _sc_plain_call()(
        user_embedding.astype(bf16), dish_embedding.astype(bf16),
        store_embedding.astype(bf16), category_embedding,
        user_user_id.astype(i32), item_dish_id.astype(i32),
        item_store_id.astype(i32), item_category.astype(i32),
        user_age.astype(f32), user_time_of_day.astype(f32),
        item_price.astype(f32), item_rating.astype(f32),
        item_time_of_day.astype(f32), user_gender.astype(i32),
        user_day_of_week.astype(i32), item_day_of_week.astype(i32))

    perm = jnp.array(PERM32)
    row0 = tag_embedding[0:1, perm].astype(jnp.bfloat16).astype(jnp.float32)
    idxu = list(range(208))
    for s in (0, 32, 112, 144, 176):
        idxu[s:s + 32] = [s + p for p in PERM32]
    idxi = list(range(184))
    for s in (0, 32, 64, 96):
        idxi[s:s + 32] = [s + p for p in PERM32]
    Wu_p = user_proj_W[jnp.array(idxu), :]
    Wi_p = item_proj_W[jnp.array(idxi), :]
    u_in, i_in = 208, 184
    weights = dict(
        Wu=(Wu_p, (u_in, D)), bu=(user_proj_b.reshape(1, D), (1, D)),
        Wi=(Wi_p, (i_in, D)), bi=(item_proj_b.reshape(1, D), (1, D)),
        gemb=(user_gender_emb, (3, 16)), uday=(user_day_emb, (7, 8)),
        iday=(dish_day_emb, (7, 8)),
        ageW=(user_age_W, (1, 16)), ageb=(user_age_b.reshape(1, 16), (1, 16)),
        utW=(user_time_W, (1, 8)), utb=(user_time_b.reshape(1, 8), (1, 8)),
        prW=(dish_price_W, (1, 16)), prb=(dish_price_b.reshape(1, 16), (1, 16)),
        rtW=(dish_rating_W, (1, 8)), rtb=(dish_rating_b.reshape(1, 8), (1, 8)),
        itW=(dish_time_W, (1, 8)), itb=(dish_time_b.reshape(1, 8), (1, 8)),
        row0=(row0, (1, 32)),
    )

    in_specs = ([_row_spec(128)] * 4
                + [_full_spec(s) for (_, s) in weights.values()])

    un, ivn, dotv = pl.pallas_call(
        _tc_body,
        grid=(B // BN,),
        in_specs=in_specs,
        out_specs=[_row_spec(D), _row_spec(D), _row_spec(1)],
        out_shape=[
            jax.ShapeDtypeStruct((B, D), f32),
            jax.ShapeDtypeStruct((B, D), f32),
            jax.ShapeDtypeStruct((B, 1), f32),
        ],
    )(o1, oc, o2, o3, *[w for (w, _) in weights.values()])

    return un, ivn, dotv.reshape(B)


# speedup vs baseline: 1.3408x; 1.3408x over previous
import functools

import jax
import jax.numpy as jnp
from jax import lax
from jax.experimental import pallas as pl
from jax.experimental.pallas import tpu as pltpu
from jax.experimental.pallas import tpu_sc as plsc

B = 16384
D = 64
T = 20
NC, NS = 2, 16
NW = NC * NS
BC = B // NW
CP = 128
NCP = BC // CP
PERM32 = list(range(0, 32, 2)) + list(range(1, 32, 2))
CG = 128
NCG = BC // CG
LANES = 16
NG = BC // LANES

_MESH = dict(core_axis_name="c", subcore_axis_name="s",
             num_cores=NC, num_subcores=NS)
_PARAMS = dict(use_tc_tiling_on_sc=False, needs_layout_passes=False)


def _wid_base():
    wid = lax.axis_index("s") * NC + lax.axis_index("c")
    return wid * BC


def _sc_pools_body(tag_t, likedT, dislikedT, allergyT, tagsT,
                   out1, outc,
                   idx_v, rows_v, pout_v, misc_v, sem_g):
    base = _wid_base()
    f32 = jnp.float32
    i32 = jnp.int32

    def pool(pi, idxT_hbm):
        def load_and_fire(c):
            p = lax.rem(c, 2)
            off = pl.multiple_of(base + c * CP, CP)
            pltpu.sync_copy(idxT_hbm.at[:, pl.ds(off, CP)], idx_v.at[p])

            def fire(k, carry):
                pltpu.make_async_copy(tag_t.at[idx_v.at[p, k]],
                                      rows_v.at[p, k], sem_g).start()
                return carry
            lax.fori_loop(0, T, fire, 0)

        def reduce_and_out(c):
            p = lax.rem(c, 2)
            off = pl.multiple_of(base + c * CP, CP)

            def drain(k, carry):
                pltpu.make_async_copy(tag_t.at[idx_v.at[p, 0]],
                                      rows_v.at[p, 0], sem_g).wait()
                return carry
            lax.fori_loop(0, T, drain, 0)

            def red(i, carry):
                s0 = jnp.zeros((LANES,), f32)
                s1 = jnp.zeros((LANES,), f32)
                for k in range(T):
                    x = rows_v[p, k, i, :]
                    a, b = plsc.unpack(x, format=plsc.PackFormat.INTERLEAVED)
                    s0 = s0 + a
                    s1 = s1 + b
                pout_v[i, pl.ds(0, LANES)] = s0
                pout_v[i, pl.ds(LANES, LANES)] = s1
                return carry
            lax.fori_loop(0, CP, red, 0)
            pltpu.sync_copy(pout_v,
                            out1.at[pl.ds(off, CP), pl.ds(32 * pi, 32)])

            colv = jnp.full((LANES,), pi, i32)

            def cnt(g, carry):
                sl = pl.ds(g * LANES, LANES)
                s = jnp.zeros((LANES,), f32)
                for k in range(T):
                    s = s + (idx_v[p, k, sl] != 0).astype(f32)
                rowv = lax.iota(i32, LANES) + (c * CP + g * LANES)
                plsc.store_scatter(misc_v, [rowv, colv], s)
                return carry
            lax.fori_loop(0, CP // LANES, cnt, 0)

        load_and_fire(0)
        load_and_fire(1)

        def body(c, carry):
            reduce_and_out(c)
            @pl.when(c + 2 < NCP)
            def _():
                load_and_fire(c + 2)
            return carry
        lax.fori_loop(0, NCP, body, 0)

    pool(0, likedT)
    pool(1, dislikedT)
    pool(2, allergyT)
    pool(3, tagsT)

    pltpu.sync_copy(misc_v, outc.at[pl.ds(base, BC), pl.ds(0, 16)])


def _sc_plain_body(user_t, dish_t, store_t, cat_t,
                   uid, did, sid, cid,
                   age, utod, price, rating, itod, gid, udy, idy,
                   out2, out3,
                   misc_v, tmpf_v, tmpi_v, gidx_v, g64_v, g32_v, g16_v,
                   sem_p):
    base = _wid_base()
    f32 = jnp.float32
    i32 = jnp.int32

    def pack_scalar(col, src_hbm, is_int):
        tmp = tmpi_v if is_int else tmpf_v
        pltpu.sync_copy(src_hbm.at[pl.ds(base, BC)], tmp)
        colv = jnp.full((LANES,), col, i32)

        def body(g, carry):
            vals = tmp[pl.ds(g * LANES, LANES)]
            if is_int:
                vals = vals.astype(f32)
            rowv = lax.iota(i32, LANES) + g * LANES
            plsc.store_scatter(misc_v, [rowv, colv], vals)
            return carry
        lax.fori_loop(0, NG, body, 0)

    for col, (src, is_int) in enumerate([
            (age, False), (utod, False), (price, False), (rating, False),
            (itod, False), (gid, True), (udy, True), (idy, True)]):
        pack_scalar(col, src, is_int)

    def plain(table, ids_hbm, out_hbm, col0, width, buf):
        def stage(c):
            p = lax.rem(c, 2)
            off = pl.multiple_of(base + c * CG, CG)
            pltpu.sync_copy(ids_hbm.at[pl.ds(off, CG)], gidx_v.at[p])
            pltpu.make_async_copy(table.at[gidx_v.at[p]], buf.at[p],
                                  sem_p).start()

        def drain(c):
            p = lax.rem(c, 2)
            off = pl.multiple_of(base + c * CG, CG)
            pltpu.make_async_copy(table.at[gidx_v.at[p]], buf.at[p],
                                  sem_p).wait()
            pltpu.sync_copy(buf.at[p],
                            out_hbm.at[pl.ds(off, CG), pl.ds(col0, width)])

        stage(0)

        def body(c, carry):
            @pl.when(c + 1 < NCG)
            def _():
                stage(c + 1)
            drain(c)
            return carry
        lax.fori_loop(0, NCG, body, 0)

    plain(user_t, uid, out2, 0, 64, g64_v)
    plain(dish_t, did, out2, 64, 64, g64_v)
    plain(store_t, sid, out3, 0, 32, g32_v)
    plain(cat_t, cid, out3, 32, 16, g16_v)

    pltpu.sync_copy(misc_v, out3.at[pl.ds(base, BC), pl.ds(48, 16)])


@functools.cache
def _sc_pools_call():
    return pl.kernel(
        _sc_pools_body,
        out_type=(
            jax.ShapeDtypeStruct((B, 128), jnp.float32),
            jax.ShapeDtypeStruct((B, 128), jnp.float32),
        ),
        mesh=plsc.VectorSubcoreMesh(**_MESH),
        scratch_types=[
            pltpu.VMEM((2, T, CP), jnp.int32),
            pltpu.VMEM((2, T, CP, 32), jnp.bfloat16),
            pltpu.VMEM((CP, 32), jnp.float32),
            pltpu.VMEM((BC, 16), jnp.float32),
            pltpu.SemaphoreType.DMA,
        ],
        compiler_params=pltpu.CompilerParams(**_PARAMS),
    )


@functools.cache
def _sc_plain_call():
    return pl.kernel(
        _sc_plain_body,
        out_type=(
            jax.ShapeDtypeStruct((B, 128), jnp.float32),
            jax.ShapeDtypeStruct((B, 128), jnp.float32),
        ),
        mesh=plsc.VectorSubcoreMesh(**_MESH),
        scratch_types=[
            pltpu.VMEM((BC, 16), jnp.float32),
            pltpu.VMEM((BC,), jnp.float32),
            pltpu.VMEM((BC,), jnp.int32),
            pltpu.VMEM((2, CG), jnp.int32),
            pltpu.VMEM((2, CG, 64), jnp.float32),
            pltpu.VMEM((2, CG, 32), jnp.float32),
            pltpu.VMEM((2, CG, 16), jnp.float32),
            pltpu.SemaphoreType.DMA,
        ],
        compiler_params=pltpu.CompilerParams(**_PARAMS),
    )


BN = 1024


def _tc_body(o1, oc, o2, o3,
             Wu, bu, Wi, bi, gemb, uday, iday,
             ageW, ageb, utW, utb, prW, prb, rtW, rtb, itW, itb, row0,
             uo, io, doto):
    f32 = jnp.float32
    dot = functools.partial(lax.dot, preferred_element_type=f32)
    r0 = row0[...]
    o1_ = o1[...]
    oc_ = oc[...]
    o2_ = o2[...]
    o3_ = o3[...]

    def pool(acc, cnt):
        return (acc - (float(T) - cnt) * r0) / (cnt + 1e-8)

    liked = pool(o1_[:, 0:32], oc_[:, 0:1])
    disl = pool(o1_[:, 32:64], oc_[:, 1:2])
    alle = pool(o1_[:, 64:96], oc_[:, 2:3])
    tagv = pool(o1_[:, 96:128], oc_[:, 3:4])
    age = o3_[:, 48:49]
    utod = o3_[:, 49:50]
    price = o3_[:, 50:51]
    rating = o3_[:, 51:52]
    itod = o3_[:, 52:53]

    def onehot(col, n):
        ci = col.astype(jnp.int32)
        return (lax.broadcasted_iota(jnp.int32, (BN, n), 1) == ci).astype(f32)

    Wu_ = Wu[...]
    u = dot(o2_[:, 0:64], Wu_[0:64])
    u += dot(liked, Wu_[112:144])
    u += dot(disl, Wu_[144:176])
    u += dot(alle, Wu_[176:208])
    u += age * dot(ageW[...], Wu_[64:80])
    u += utod * dot(utW[...], Wu_[96:104])
    u += dot(onehot(o3_[:, 53:54], 3), dot(gemb[...], Wu_[80:96]))
    u += dot(onehot(o3_[:, 54:55], 7), dot(uday[...], Wu_[104:112]))
    u += bu[...] + dot(ageb[...], Wu_[64:80]) + dot(utb[...], Wu_[96:104])
    nu = jnp.sqrt(jnp.sum(u * u, axis=1, keepdims=True))
    un = u / jnp.maximum(nu, 1e-12)

    Wi_ = Wi[...]
    iv = dot(o2_[:, 64:128], Wi_[0:64])
    iv += dot(o3_[:, 0:32], Wi_[64:96])
    iv += dot(tagv, Wi_[96:128])
    iv += dot(o3_[:, 32:48], Wi_[128:144])
    iv += price * dot(prW[...], Wi_[144:160])
    iv += rating * dot(rtW[...], Wi_[160:168])
    iv += itod * dot(itW[...], Wi_[168:176])
    iv += dot(onehot(o3_[:, 55:56], 7), dot(iday[...], Wi_[176:184]))
    iv += (bi[...] + dot(prb[...], Wi_[144:160]) + dot(rtb[...], Wi_[160:168])
           + dot(itb[...], Wi_[168:176]))
    ni = jnp.sqrt(jnp.sum(iv * iv, axis=1, keepdims=True))
    ivn = iv / jnp.maximum(ni, 1e-12)

    uo[...] = un
    io[...] = ivn
    doto[...] = jnp.sum(un * ivn, axis=1, keepdims=True)


def _row_spec(k):
    return pl.BlockSpec((BN, k), lambda i: (i, 0))


def _full_spec(shape):
    return pl.BlockSpec(shape, lambda i: (0,) * len(shape))


def kernel(user_user_id, user_age, user_gender, user_time_of_day,
           user_day_of_week, user_liked_tags, user_disliked_tags,
           user_allergy_tags, item_dish_id, item_store_id, item_category,
           item_tags, item_price, item_rating, item_time_of_day,
           item_day_of_week, user_embedding, user_age_W, user_age_b,
           user_gender_emb, user_time_W, user_time_b, user_day_emb,
           dish_embedding, store_embedding, category_embedding,
           dish_price_W, dish_price_b, dish_rating_W, dish_rating_b,
           dish_time_W, dish_time_b, dish_day_emb, tag_embedding,
           user_proj_W, user_proj_b, item_proj_W, item_proj_b):
    i32 = jnp.int32
    f32 = jnp.float32

    o1, oc = _sc_pools_call()(
        tag_embedding.astype(jnp.bfloat16),
        user_liked_tags.astype(i32).T, user_disliked_tags.astype(i32).T,
        user_allergy_tags.astype(i32).T, item_tags.astype(i32).T)

    o2, o3 = _sc_plain_call()(
        user_embedding, dish_embedding, store_embedding, category_embedding,
        user_user_id.astype(i32), item_dish_id.astype(i32),
        item_store_id.astype(i32), item_category.astype(i32),
        user_age.astype(f32), user_time_of_day.astype(f32),
        item_price.astype(f32), item_rating.astype(f32),
        item_time_of_day.astype(f32), user_gender.astype(i32),
        user_day_of_week.astype(i32), item_day_of_week.astype(i32))

    perm = jnp.array(PERM32)
    row0 = tag_embedding[0:1, perm].astype(jnp.bfloat16).astype(jnp.float32)
    idxu = list(range(208))
    for s in (112, 144, 176):
        idxu[s:s + 32] = [s + p for p in PERM32]
    idxi = list(range(184))
    idxi[96:128] = [96 + p for p in PERM32]
    Wu_p = user_proj_W[jnp.array(idxu), :]
    Wi_p = item_proj_W[jnp.array(idxi), :]
    u_in, i_in = 208, 184
    weights = dict(
        Wu=(Wu_p, (u_in, D)), bu=(user_proj_b.reshape(1, D), (1, D)),
        Wi=(Wi_p, (i_in, D)), bi=(item_proj_b.reshape(1, D), (1, D)),
        gemb=(user_gender_emb, (3, 16)), uday=(user_day_emb, (7, 8)),
        iday=(dish_day_emb, (7, 8)),
        ageW=(user_age_W, (1, 16)), ageb=(user_age_b.reshape(1, 16), (1, 16)),
        utW=(user_time_W, (1, 8)), utb=(user_time_b.reshape(1, 8), (1, 8)),
        prW=(dish_price_W, (1, 16)), prb=(dish_price_b.reshape(1, 16), (1, 16)),
        rtW=(dish_rating_W, (1, 8)), rtb=(dish_rating_b.reshape(1, 8), (1, 8)),
        itW=(dish_time_W, (1, 8)), itb=(dish_time_b.reshape(1, 8), (1, 8)),
        row0=(row0, (1, 32)),
    )

    in_specs = ([_row_spec(128)] * 4
                + [_full_spec(s) for (_, s) in weights.values()])

    un, ivn, dotv = pl.pallas_call(
        _tc_body,
        grid=(B // BN,),
        in_specs=in_specs,
        out_specs=[_row_spec(D), _row_spec(D), _row_spec(1)],
        out_shape=[
            jax.ShapeDtypeStruct((B, D), f32),
            jax.ShapeDtypeStruct((B, D), f32),
            jax.ShapeDtypeStruct((B, 1), f32),
        ],
    )(o1, oc, o2, o3, *[w for (w, _) in weights.values()])

    return un, ivn, dotv.reshape(B)


# speedup vs baseline: 1.3468x; 1.0044x over previous
import functools

import jax
import jax.numpy as jnp
from jax import lax
from jax.experimental import pallas as pl
from jax.experimental.pallas import tpu as pltpu
from jax.experimental.pallas import tpu_sc as plsc

B = 16384
D = 64
T = 20
NC, NS = 2, 16
NW = NC * NS
BC = B // NW
CP = 128
NCP = BC // CP
PERM32 = list(range(0, 32, 2)) + list(range(1, 32, 2))
CG = 128
NCG = BC // CG
LANES = 16
NG = BC // LANES

_MESH = dict(core_axis_name="c", subcore_axis_name="s",
             num_cores=NC, num_subcores=NS)
_PARAMS = dict(use_tc_tiling_on_sc=False, needs_layout_passes=False)


def _wid_base():
    wid = lax.axis_index("s") * NC + lax.axis_index("c")
    return wid * BC


def _sc_pools_body(tag_t, likedT, dislikedT, allergyT, tagsT,
                   out1, outc,
                   idx_v, rows_v, pout_v, misc_v, sem_g):
    base = _wid_base()
    f32 = jnp.float32
    i32 = jnp.int32

    def pool(pi, idxT_hbm):
        def load_and_fire(c):
            p = lax.rem(c, 2)
            off = pl.multiple_of(base + c * CP, CP)
            pltpu.sync_copy(idxT_hbm.at[:, pl.ds(off, CP)], idx_v.at[p])

            def fire(k, carry):
                pltpu.make_async_copy(tag_t.at[idx_v.at[p, k]],
                                      rows_v.at[p, k], sem_g).start()
                return carry
            lax.fori_loop(0, T, fire, 0)

        def reduce_and_out(c):
            p = lax.rem(c, 2)
            off = pl.multiple_of(base + c * CP, CP)

            def drain(k, carry):
                pltpu.make_async_copy(tag_t.at[idx_v.at[p, 0]],
                                      rows_v.at[p, 0], sem_g).wait()
                return carry
            lax.fori_loop(0, T, drain, 0)

            def red(i, carry):
                s0 = jnp.zeros((LANES,), f32)
                s1 = jnp.zeros((LANES,), f32)
                for k in range(T):
                    x = rows_v[p, k, i, :]
                    a, b = plsc.unpack(x, format=plsc.PackFormat.INTERLEAVED)
                    s0 = s0 + a
                    s1 = s1 + b
                pout_v[i, pl.ds(0, LANES)] = s0
                pout_v[i, pl.ds(LANES, LANES)] = s1
                return carry
            lax.fori_loop(0, CP, red, 0)
            pltpu.sync_copy(pout_v,
                            out1.at[pl.ds(off, CP), pl.ds(32 * pi, 32)])

            colv = jnp.full((LANES,), pi, i32)

            def cnt(g, carry):
                sl = pl.ds(g * LANES, LANES)
                s = jnp.zeros((LANES,), f32)
                for k in range(T):
                    s = s + (idx_v[p, k, sl] != 0).astype(f32)
                rowv = lax.iota(i32, LANES) + (c * CP + g * LANES)
                plsc.store_scatter(misc_v, [rowv, colv], s)
                return carry
            lax.fori_loop(0, CP // LANES, cnt, 0)

        load_and_fire(0)
        load_and_fire(1)

        def body(c, carry):
            reduce_and_out(c)
            @pl.when(c + 2 < NCP)
            def _():
                load_and_fire(c + 2)
            return carry
        lax.fori_loop(0, NCP, body, 0)

    pool(0, likedT)
    pool(1, dislikedT)
    pool(2, allergyT)
    pool(3, tagsT)

    pltpu.sync_copy(misc_v, outc.at[pl.ds(base, BC), pl.ds(0, 16)])


def _sc_plain_body(user_t, dish_t, store_t, cat_t,
                   uid, did, sid, cid,
                   age, utod, price, rating, itod, gid, udy, idy,
                   out2, out3,
                   misc_v, tmpf_v, tmpi_v, gidx_v, g64_v, g32_v, g16_v,
                   sem_p):
    base = _wid_base()
    f32 = jnp.float32
    i32 = jnp.int32

    def pack_scalar(col, src_hbm, is_int):
        tmp = tmpi_v if is_int else tmpf_v
        pltpu.sync_copy(src_hbm.at[pl.ds(base, BC)], tmp)
        colv = jnp.full((LANES,), col, i32)

        def body(g, carry):
            vals = tmp[pl.ds(g * LANES, LANES)]
            if is_int:
                vals = vals.astype(f32)
            rowv = lax.iota(i32, LANES) + g * LANES
            plsc.store_scatter(misc_v, [rowv, colv], vals)
            return carry
        lax.fori_loop(0, NG, body, 0)

    for col, (src, is_int) in enumerate([
            (age, False), (utod, False), (price, False), (rating, False),
            (itod, False), (gid, True), (udy, True), (idy, True)]):
        pack_scalar(col, src, is_int)

    def plain(table, ids_hbm, out_hbm, col0, width, buf):
        def stage(c):
            p = lax.rem(c, 2)
            off = pl.multiple_of(base + c * CG, CG)
            pltpu.sync_copy(ids_hbm.at[pl.ds(off, CG)], gidx_v.at[p])
            pltpu.make_async_copy(table.at[gidx_v.at[p]], buf.at[p],
                                  sem_p).start()

        def drain(c):
            p = lax.rem(c, 2)
            off = pl.multiple_of(base + c * CG, CG)
            pltpu.make_async_copy(table.at[gidx_v.at[p]], buf.at[p],
                                  sem_p).wait()
            pltpu.sync_copy(buf.at[p],
                            out_hbm.at[pl.ds(off, CG), pl.ds(col0, width)])

        stage(0)

        def body(c, carry):
            @pl.when(c + 1 < NCG)
            def _():
                stage(c + 1)
            drain(c)
            return carry
        lax.fori_loop(0, NCG, body, 0)

    plain(user_t, uid, out2, 0, 64, g64_v)
    plain(dish_t, did, out2, 64, 64, g64_v)
    plain(store_t, sid, out3, 0, 32, g32_v)
    plain(cat_t, cid, out3, 32, 16, g16_v)

    pltpu.sync_copy(misc_v, out3.at[pl.ds(base, BC), pl.ds(48, 16)])


@functools.cache
def _sc_pools_call():
    return pl.kernel(
        _sc_pools_body,
        out_type=(
            jax.ShapeDtypeStruct((B, 128), jnp.float32),
            jax.ShapeDtypeStruct((B, 128), jnp.float32),
        ),
        mesh=plsc.VectorSubcoreMesh(**_MESH),
        scratch_types=[
            pltpu.VMEM((2, T, CP), jnp.int32),
            pltpu.VMEM((2, T, CP, 32), jnp.bfloat16),
            pltpu.VMEM((CP, 32), jnp.float32),
            pltpu.VMEM((BC, 16), jnp.float32),
            pltpu.SemaphoreType.DMA,
        ],
        compiler_params=pltpu.CompilerParams(**_PARAMS),
    )


@functools.cache
def _sc_plain_call():
    return pl.kernel(
        _sc_plain_body,
        out_type=(
            jax.ShapeDtypeStruct((B, 128), jnp.float32),
            jax.ShapeDtypeStruct((B, 128), jnp.float32),
        ),
        mesh=plsc.VectorSubcoreMesh(**_MESH),
        scratch_types=[
            pltpu.VMEM((BC, 16), jnp.float32),
            pltpu.VMEM((BC,), jnp.float32),
            pltpu.VMEM((BC,), jnp.int32),
            pltpu.VMEM((2, CG), jnp.int32),
            pltpu.VMEM((2, CG, 64), jnp.float32),
            pltpu.VMEM((2, CG, 32), jnp.float32),
            pltpu.VMEM((2, CG, 16), jnp.float32),
            pltpu.SemaphoreType.DMA,
        ],
        compiler_params=pltpu.CompilerParams(**_PARAMS),
    )


BN = 2048


def _tc_body(o1, oc, o2, o3,
             Wu, bu, Wi, bi, gemb, uday, iday,
             ageW, ageb, utW, utb, prW, prb, rtW, rtb, itW, itb, row0,
             uo, io, doto):
    f32 = jnp.float32
    dot = functools.partial(lax.dot, preferred_element_type=f32)
    r0 = row0[...]
    o1_ = o1[...]
    oc_ = oc[...]
    o2_ = o2[...]
    o3_ = o3[...]

    def pool(acc, cnt):
        return (acc - (float(T) - cnt) * r0) / (cnt + 1e-8)

    liked = pool(o1_[:, 0:32], oc_[:, 0:1])
    disl = pool(o1_[:, 32:64], oc_[:, 1:2])
    alle = pool(o1_[:, 64:96], oc_[:, 2:3])
    tagv = pool(o1_[:, 96:128], oc_[:, 3:4])
    age = o3_[:, 48:49]
    utod = o3_[:, 49:50]
    price = o3_[:, 50:51]
    rating = o3_[:, 51:52]
    itod = o3_[:, 52:53]

    def onehot(col, n):
        ci = col.astype(jnp.int32)
        return (lax.broadcasted_iota(jnp.int32, (BN, n), 1) == ci).astype(f32)

    Wu_ = Wu[...]
    u = dot(o2_[:, 0:64], Wu_[0:64])
    u += dot(liked, Wu_[112:144])
    u += dot(disl, Wu_[144:176])
    u += dot(alle, Wu_[176:208])
    u += age * dot(ageW[...], Wu_[64:80])
    u += utod * dot(utW[...], Wu_[96:104])
    u += dot(onehot(o3_[:, 53:54], 3), dot(gemb[...], Wu_[80:96]))
    u += dot(onehot(o3_[:, 54:55], 7), dot(uday[...], Wu_[104:112]))
    u += bu[...] + dot(ageb[...], Wu_[64:80]) + dot(utb[...], Wu_[96:104])
    nu = jnp.sqrt(jnp.sum(u * u, axis=1, keepdims=True))
    un = u / jnp.maximum(nu, 1e-12)

    Wi_ = Wi[...]
    iv = dot(o2_[:, 64:128], Wi_[0:64])
    iv += dot(o3_[:, 0:32], Wi_[64:96])
    iv += dot(tagv, Wi_[96:128])
    iv += dot(o3_[:, 32:48], Wi_[128:144])
    iv += price * dot(prW[...], Wi_[144:160])
    iv += rating * dot(rtW[...], Wi_[160:168])
    iv += itod * dot(itW[...], Wi_[168:176])
    iv += dot(onehot(o3_[:, 55:56], 7), dot(iday[...], Wi_[176:184]))
    iv += (bi[...] + dot(prb[...], Wi_[144:160]) + dot(rtb[...], Wi_[160:168])
           + dot(itb[...], Wi_[168:176]))
    ni = jnp.sqrt(jnp.sum(iv * iv, axis=1, keepdims=True))
    ivn = iv / jnp.maximum(ni, 1e-12)

    uo[...] = un
    io[...] = ivn
    doto[...] = jnp.sum(un * ivn, axis=1, keepdims=True)


def _row_spec(k):
    return pl.BlockSpec((BN, k), lambda i: (i, 0))


def _full_spec(shape):
    return pl.BlockSpec(shape, lambda i: (0,) * len(shape))


def kernel(user_user_id, user_age, user_gender, user_time_of_day,
           user_day_of_week, user_liked_tags, user_disliked_tags,
           user_allergy_tags, item_dish_id, item_store_id, item_category,
           item_tags, item_price, item_rating, item_time_of_day,
           item_day_of_week, user_embedding, user_age_W, user_age_b,
           user_gender_emb, user_time_W, user_time_b, user_day_emb,
           dish_embedding, store_embedding, category_embedding,
           dish_price_W, dish_price_b, dish_rating_W, dish_rating_b,
           dish_time_W, dish_time_b, dish_day_emb, tag_embedding,
           user_proj_W, user_proj_b, item_proj_W, item_proj_b):
    i32 = jnp.int32
    f32 = jnp.float32

    o1, oc = _sc_pools_call()(
        tag_embedding.astype(jnp.bfloat16),
        user_liked_tags.astype(i32).T, user_disliked_tags.astype(i32).T,
        user_allergy_tags.astype(i32).T, item_tags.astype(i32).T)

    o2, o3 = _sc_plain_call()(
        user_embedding, dish_embedding, store_embedding, category_embedding,
        user_user_id.astype(i32), item_dish_id.astype(i32),
        item_store_id.astype(i32), item_category.astype(i32),
        user_age.astype(f32), user_time_of_day.astype(f32),
        item_price.astype(f32), item_rating.astype(f32),
        item_time_of_day.astype(f32), user_gender.astype(i32),
        user_day_of_week.astype(i32), item_day_of_week.astype(i32))

    perm = jnp.array(PERM32)
    row0 = tag_embedding[0:1, perm].astype(jnp.bfloat16).astype(jnp.float32)
    idxu = list(range(208))
    for s in (112, 144, 176):
        idxu[s:s + 32] = [s + p for p in PERM32]
    idxi = list(range(184))
    idxi[96:128] = [96 + p for p in PERM32]
    Wu_p = user_proj_W[jnp.array(idxu), :]
    Wi_p = item_proj_W[jnp.array(idxi), :]
    u_in, i_in = 208, 184
    weights = dict(
        Wu=(Wu_p, (u_in, D)), bu=(user_proj_b.reshape(1, D), (1, D)),
        Wi=(Wi_p, (i_in, D)), bi=(item_proj_b.reshape(1, D), (1, D)),
        gemb=(user_gender_emb, (3, 16)), uday=(user_day_emb, (7, 8)),
        iday=(dish_day_emb, (7, 8)),
        ageW=(user_age_W, (1, 16)), ageb=(user_age_b.reshape(1, 16), (1, 16)),
        utW=(user_time_W, (1, 8)), utb=(user_time_b.reshape(1, 8), (1, 8)),
        prW=(dish_price_W, (1, 16)), prb=(dish_price_b.reshape(1, 16), (1, 16)),
        rtW=(dish_rating_W, (1, 8)), rtb=(dish_rating_b.reshape(1, 8), (1, 8)),
        itW=(dish_time_W, (1, 8)), itb=(dish_time_b.reshape(1, 8), (1, 8)),
        row0=(row0, (1, 32)),
    )

    in_specs = ([_row_spec(128)] * 4
                + [_full_spec(s) for (_, s) in weights.values()])

    un, ivn, dotv = pl.pallas_call(
        _tc_body,
        grid=(B // BN,),
        in_specs=in_specs,
        out_specs=[_row_spec(D), _row_spec(D), _row_spec(1)],
        out_shape=[
            jax.ShapeDtypeStruct((B, D), f32),
            jax.ShapeDtypeStruct((B, D), f32),
            jax.ShapeDtypeStruct((B, 1), f32),
        ],
    )(o1, oc, o2, o3, *[w for (w, _) in weights.values()])

    return un, ivn, dotv.reshape(B)


# speedup vs baseline: 1.3514x; 1.0034x over previous
import functools

import jax
import jax.numpy as jnp
from jax import lax
from jax.experimental import pallas as pl
from jax.experimental.pallas import tpu as pltpu
from jax.experimental.pallas import tpu_sc as plsc

B = 16384
D = 64
T = 20
NC, NS = 2, 16
NW = NC * NS
BC = B // NW
CP = 128
NCP = BC // CP
PERM32 = list(range(0, 32, 2)) + list(range(1, 32, 2))
CG = 128
NCG = BC // CG
LANES = 16
NG = BC // LANES

_MESH = dict(core_axis_name="c", subcore_axis_name="s",
             num_cores=NC, num_subcores=NS)
_PARAMS = dict(use_tc_tiling_on_sc=False, needs_layout_passes=False)


def _wid_base():
    wid = lax.axis_index("s") * NC + lax.axis_index("c")
    return wid * BC


def _sc_pools_body(tag_t, likedT, dislikedT, allergyT, tagsT,
                   out1, outc,
                   idx_v, rows_v, pout_v, misc_v, sem_g):
    base = _wid_base()
    f32 = jnp.float32
    i32 = jnp.int32

    def pool(pi, idxT_hbm):
        def load_and_fire(c):
            p = lax.rem(c, 2)
            off = pl.multiple_of(base + c * CP, CP)
            pltpu.sync_copy(idxT_hbm.at[:, pl.ds(off, CP)], idx_v.at[p])

            def fire(k, carry):
                pltpu.make_async_copy(tag_t.at[idx_v.at[p, k]],
                                      rows_v.at[p, k], sem_g).start()
                return carry
            lax.fori_loop(0, T, fire, 0)

        def reduce_and_out(c):
            p = lax.rem(c, 2)
            off = pl.multiple_of(base + c * CP, CP)

            def drain(k, carry):
                pltpu.make_async_copy(tag_t.at[idx_v.at[p, 0]],
                                      rows_v.at[p, 0], sem_g).wait()
                return carry
            lax.fori_loop(0, T, drain, 0)

            def red(i, carry):
                s0 = jnp.zeros((LANES,), f32)
                s1 = jnp.zeros((LANES,), f32)
                for k in range(T):
                    x = rows_v[p, k, i, :]
                    a, b = plsc.unpack(x, format=plsc.PackFormat.INTERLEAVED)
                    s0 = s0 + a
                    s1 = s1 + b
                pout_v[i, pl.ds(0, LANES)] = s0
                pout_v[i, pl.ds(LANES, LANES)] = s1
                return carry
            lax.fori_loop(0, CP, red, 0)
            pltpu.sync_copy(pout_v,
                            out1.at[pl.ds(off, CP), pl.ds(32 * pi, 32)])

            colv = jnp.full((LANES,), pi, i32)

            def cnt(g, carry):
                sl = pl.ds(g * LANES, LANES)
                s = jnp.zeros((LANES,), f32)
                for k in range(T):
                    s = s + (idx_v[p, k, sl] != 0).astype(f32)
                rowv = lax.iota(i32, LANES) + (c * CP + g * LANES)
                plsc.store_scatter(misc_v, [rowv, colv], s)
                return carry
            lax.fori_loop(0, CP // LANES, cnt, 0)

        load_and_fire(0)
        load_and_fire(1)

        def body(c, carry):
            reduce_and_out(c)
            @pl.when(c + 2 < NCP)
            def _():
                load_and_fire(c + 2)
            return carry
        lax.fori_loop(0, NCP, body, 0)

    pool(0, likedT)
    pool(1, dislikedT)
    pool(2, allergyT)
    pool(3, tagsT)

    pltpu.sync_copy(misc_v, outc.at[pl.ds(base, BC), pl.ds(0, 16)])


def _sc_plain_body(user_t, dish_t, store_t, cat_t,
                   uid, did, sid, cid,
                   age, utod, price, rating, itod, gid, udy, idy,
                   out2, out3,
                   misc_v, tmpf_v, tmpi_v, gidx_v, g64_v, g32_v, g16_v,
                   sem_p):
    base = _wid_base()
    f32 = jnp.float32
    i32 = jnp.int32

    def pack_scalar(col, src_hbm, is_int):
        tmp = tmpi_v if is_int else tmpf_v
        pltpu.sync_copy(src_hbm.at[pl.ds(base, BC)], tmp)
        colv = jnp.full((LANES,), col, i32)

        def body(g, carry):
            vals = tmp[pl.ds(g * LANES, LANES)]
            if is_int:
                vals = vals.astype(f32)
            rowv = lax.iota(i32, LANES) + g * LANES
            plsc.store_scatter(misc_v, [rowv, colv], vals)
            return carry
        lax.fori_loop(0, NG, body, 0)

    for col, (src, is_int) in enumerate([
            (age, False), (utod, False), (price, False), (rating, False),
            (itod, False), (gid, True), (udy, True), (idy, True)]):
        pack_scalar(col, src, is_int)

    def plain(table, ids_hbm, out_hbm, col0, width, buf):
        def stage(c):
            p = lax.rem(c, 2)
            off = pl.multiple_of(base + c * CG, CG)
            pltpu.sync_copy(ids_hbm.at[pl.ds(off, CG)], gidx_v.at[p])
            pltpu.make_async_copy(table.at[gidx_v.at[p]], buf.at[p],
                                  sem_p).start()

        def drain(c):
            p = lax.rem(c, 2)
            off = pl.multiple_of(base + c * CG, CG)
            pltpu.make_async_copy(table.at[gidx_v.at[p]], buf.at[p],
                                  sem_p).wait()
            pltpu.sync_copy(buf.at[p],
                            out_hbm.at[pl.ds(off, CG), pl.ds(col0, width)])

        stage(0)

        def body(c, carry):
            @pl.when(c + 1 < NCG)
            def _():
                stage(c + 1)
            drain(c)
            return carry
        lax.fori_loop(0, NCG, body, 0)

    plain(user_t, uid, out2, 0, 64, g64_v)
    plain(dish_t, did, out2, 64, 64, g64_v)
    plain(store_t, sid, out3, 0, 32, g32_v)
    plain(cat_t, cid, out3, 32, 16, g16_v)

    pltpu.sync_copy(misc_v, out3.at[pl.ds(base, BC), pl.ds(48, 16)])


@functools.cache
def _sc_pools_call():
    return pl.kernel(
        _sc_pools_body,
        out_type=(
            jax.ShapeDtypeStruct((B, 128), jnp.float32),
            jax.ShapeDtypeStruct((B, 128), jnp.float32),
        ),
        mesh=plsc.VectorSubcoreMesh(**_MESH),
        scratch_types=[
            pltpu.VMEM((2, T, CP), jnp.int32),
            pltpu.VMEM((2, T, CP, 32), jnp.bfloat16),
            pltpu.VMEM((CP, 32), jnp.float32),
            pltpu.VMEM((BC, 16), jnp.float32),
            pltpu.SemaphoreType.DMA,
        ],
        compiler_params=pltpu.CompilerParams(**_PARAMS),
    )


@functools.cache
def _sc_plain_call():
    return pl.kernel(
        _sc_plain_body,
        out_type=(
            jax.ShapeDtypeStruct((B, 128), jnp.float32),
            jax.ShapeDtypeStruct((B, 128), jnp.float32),
        ),
        mesh=plsc.VectorSubcoreMesh(**_MESH),
        scratch_types=[
            pltpu.VMEM((BC, 16), jnp.float32),
            pltpu.VMEM((BC,), jnp.float32),
            pltpu.VMEM((BC,), jnp.int32),
            pltpu.VMEM((2, CG), jnp.int32),
            pltpu.VMEM((2, CG, 64), jnp.float32),
            pltpu.VMEM((2, CG, 32), jnp.float32),
            pltpu.VMEM((2, CG, 16), jnp.float32),
            pltpu.SemaphoreType.DMA,
        ],
        compiler_params=pltpu.CompilerParams(**_PARAMS),
    )


BN = 4096


def _tc_body(o1, oc, o2, o3,
             Wu, bu, Wi, bi, gemb, uday, iday,
             ageW, ageb, utW, utb, prW, prb, rtW, rtb, itW, itb, row0,
             uo, io, doto):
    f32 = jnp.float32
    dot = functools.partial(lax.dot, preferred_element_type=f32)
    r0 = row0[...]
    o1_ = o1[...]
    oc_ = oc[...]
    o2_ = o2[...]
    o3_ = o3[...]

    def pool(acc, cnt):
        return (acc - (float(T) - cnt) * r0) / (cnt + 1e-8)

    liked = pool(o1_[:, 0:32], oc_[:, 0:1])
    disl = pool(o1_[:, 32:64], oc_[:, 1:2])
    alle = pool(o1_[:, 64:96], oc_[:, 2:3])
    tagv = pool(o1_[:, 96:128], oc_[:, 3:4])
    age = o3_[:, 48:49]
    utod = o3_[:, 49:50]
    price = o3_[:, 50:51]
    rating = o3_[:, 51:52]
    itod = o3_[:, 52:53]

    def onehot(col, n):
        ci = col.astype(jnp.int32)
        return (lax.broadcasted_iota(jnp.int32, (BN, n), 1) == ci).astype(f32)

    Wu_ = Wu[...]
    u = dot(o2_[:, 0:64], Wu_[0:64])
    u += dot(liked, Wu_[112:144])
    u += dot(disl, Wu_[144:176])
    u += dot(alle, Wu_[176:208])
    u += age * dot(ageW[...], Wu_[64:80])
    u += utod * dot(utW[...], Wu_[96:104])
    u += dot(onehot(o3_[:, 53:54], 3), dot(gemb[...], Wu_[80:96]))
    u += dot(onehot(o3_[:, 54:55], 7), dot(uday[...], Wu_[104:112]))
    u += bu[...] + dot(ageb[...], Wu_[64:80]) + dot(utb[...], Wu_[96:104])
    nu = jnp.sqrt(jnp.sum(u * u, axis=1, keepdims=True))
    un = u / jnp.maximum(nu, 1e-12)

    Wi_ = Wi[...]
    iv = dot(o2_[:, 64:128], Wi_[0:64])
    iv += dot(o3_[:, 0:32], Wi_[64:96])
    iv += dot(tagv, Wi_[96:128])
    iv += dot(o3_[:, 32:48], Wi_[128:144])
    iv += price * dot(prW[...], Wi_[144:160])
    iv += rating * dot(rtW[...], Wi_[160:168])
    iv += itod * dot(itW[...], Wi_[168:176])
    iv += dot(onehot(o3_[:, 55:56], 7), dot(iday[...], Wi_[176:184]))
    iv += (bi[...] + dot(prb[...], Wi_[144:160]) + dot(rtb[...], Wi_[160:168])
           + dot(itb[...], Wi_[168:176]))
    ni = jnp.sqrt(jnp.sum(iv * iv, axis=1, keepdims=True))
    ivn = iv / jnp.maximum(ni, 1e-12)

    uo[...] = un
    io[...] = ivn
    doto[...] = jnp.sum(un * ivn, axis=1, keepdims=True)


def _row_spec(k):
    return pl.BlockSpec((BN, k), lambda i: (i, 0))


def _full_spec(shape):
    return pl.BlockSpec(shape, lambda i: (0,) * len(shape))


def kernel(user_user_id, user_age, user_gender, user_time_of_day,
           user_day_of_week, user_liked_tags, user_disliked_tags,
           user_allergy_tags, item_dish_id, item_store_id, item_category,
           item_tags, item_price, item_rating, item_time_of_day,
           item_day_of_week, user_embedding, user_age_W, user_age_b,
           user_gender_emb, user_time_W, user_time_b, user_day_emb,
           dish_embedding, store_embedding, category_embedding,
           dish_price_W, dish_price_b, dish_rating_W, dish_rating_b,
           dish_time_W, dish_time_b, dish_day_emb, tag_embedding,
           user_proj_W, user_proj_b, item_proj_W, item_proj_b):
    i32 = jnp.int32
    f32 = jnp.float32

    o1, oc = _sc_pools_call()(
        tag_embedding.astype(jnp.bfloat16),
        user_liked_tags.astype(i32).T, user_disliked_tags.astype(i32).T,
        user_allergy_tags.astype(i32).T, item_tags.astype(i32).T)

    o2, o3 = _sc_plain_call()(
        user_embedding, dish_embedding, store_embedding, category_embedding,
        user_user_id.astype(i32), item_dish_id.astype(i32),
        item_store_id.astype(i32), item_category.astype(i32),
        user_age.astype(f32), user_time_of_day.astype(f32),
        item_price.astype(f32), item_rating.astype(f32),
        item_time_of_day.astype(f32), user_gender.astype(i32),
        user_day_of_week.astype(i32), item_day_of_week.astype(i32))

    perm = jnp.array(PERM32)
    row0 = tag_embedding[0:1, perm].astype(jnp.bfloat16).astype(jnp.float32)
    idxu = list(range(208))
    for s in (112, 144, 176):
        idxu[s:s + 32] = [s + p for p in PERM32]
    idxi = list(range(184))
    idxi[96:128] = [96 + p for p in PERM32]
    Wu_p = user_proj_W[jnp.array(idxu), :]
    Wi_p = item_proj_W[jnp.array(idxi), :]
    u_in, i_in = 208, 184
    weights = dict(
        Wu=(Wu_p, (u_in, D)), bu=(user_proj_b.reshape(1, D), (1, D)),
        Wi=(Wi_p, (i_in, D)), bi=(item_proj_b.reshape(1, D), (1, D)),
        gemb=(user_gender_emb, (3, 16)), uday=(user_day_emb, (7, 8)),
        iday=(dish_day_emb, (7, 8)),
        ageW=(user_age_W, (1, 16)), ageb=(user_age_b.reshape(1, 16), (1, 16)),
        utW=(user_time_W, (1, 8)), utb=(user_time_b.reshape(1, 8), (1, 8)),
        prW=(dish_price_W, (1, 16)), prb=(dish_price_b.reshape(1, 16), (1, 16)),
        rtW=(dish_rating_W, (1, 8)), rtb=(dish_rating_b.reshape(1, 8), (1, 8)),
        itW=(dish_time_W, (1, 8)), itb=(dish_time_b.reshape(1, 8), (1, 8)),
        row0=(row0, (1, 32)),
    )

    in_specs = ([_row_spec(128)] * 4
                + [_full_spec(s) for (_, s) in weights.values()])

    un, ivn, dotv = pl.pallas_call(
        _tc_body,
        grid=(B // BN,),
        in_specs=in_specs,
        out_specs=[_row_spec(D), _row_spec(D), _row_spec(1)],
        out_shape=[
            jax.ShapeDtypeStruct((B, D), f32),
            jax.ShapeDtypeStruct((B, D), f32),
            jax.ShapeDtypeStruct((B, 1), f32),
        ],
    )(o1, oc, o2, o3, *[w for (w, _) in weights.values()])

    return un, ivn, dotv.reshape(B)


# speedup vs baseline: 1.3528x; 1.0010x over previous
import functools

import jax
import jax.numpy as jnp
from jax import lax
from jax.experimental import pallas as pl
from jax.experimental.pallas import tpu as pltpu
from jax.experimental.pallas import tpu_sc as plsc

B = 16384
D = 64
T = 20
NC, NS = 2, 16
NW = NC * NS
BC = B // NW
CP = 128
NCP = BC // CP
PERM32 = list(range(0, 32, 2)) + list(range(1, 32, 2))
CG = 128
NCG = BC // CG
LANES = 16
NG = BC // LANES

_MESH = dict(core_axis_name="c", subcore_axis_name="s",
             num_cores=NC, num_subcores=NS)
_PARAMS = dict(use_tc_tiling_on_sc=False, needs_layout_passes=False)


def _wid_base():
    wid = lax.axis_index("s") * NC + lax.axis_index("c")
    return wid * BC


def _sc_pools_body(tag_t, likedT, dislikedT, allergyT, tagsT,
                   out1, outc,
                   idx_v, rows_v, pout_v, misc_v, sem_g):
    base = _wid_base()
    f32 = jnp.float32
    i32 = jnp.int32

    def pool(pi, idxT_hbm):
        def load_and_fire(c):
            p = lax.rem(c, 2)
            off = pl.multiple_of(base + c * CP, CP)
            pltpu.sync_copy(idxT_hbm.at[:, pl.ds(off, CP)], idx_v.at[p])

            def fire(k, carry):
                pltpu.make_async_copy(tag_t.at[idx_v.at[p, k]],
                                      rows_v.at[p, k], sem_g).start()
                return carry
            lax.fori_loop(0, T, fire, 0)

        def reduce_and_out(c):
            p = lax.rem(c, 2)
            off = pl.multiple_of(base + c * CP, CP)

            def drain(k, carry):
                pltpu.make_async_copy(tag_t.at[idx_v.at[p, 0]],
                                      rows_v.at[p, 0], sem_g).wait()
                return carry
            lax.fori_loop(0, T, drain, 0)

            @plsc.parallel_loop(0, CP, unroll=2)
            def red(i):
                s0 = jnp.zeros((LANES,), f32)
                s1 = jnp.zeros((LANES,), f32)
                for k in range(T):
                    x = rows_v[p, k, i, :]
                    a, b = plsc.unpack(x, format=plsc.PackFormat.INTERLEAVED)
                    s0 = s0 + a
                    s1 = s1 + b
                pout_v[i, pl.ds(0, LANES)] = s0
                pout_v[i, pl.ds(LANES, LANES)] = s1
            pltpu.sync_copy(pout_v,
                            out1.at[pl.ds(off, CP), pl.ds(32 * pi, 32)])

            colv = jnp.full((LANES,), pi, i32)

            def cnt(g, carry):
                sl = pl.ds(g * LANES, LANES)
                s = jnp.zeros((LANES,), f32)
                for k in range(T):
                    s = s + (idx_v[p, k, sl] != 0).astype(f32)
                rowv = lax.iota(i32, LANES) + (c * CP + g * LANES)
                plsc.store_scatter(misc_v, [rowv, colv], s)
                return carry
            lax.fori_loop(0, CP // LANES, cnt, 0)

        load_and_fire(0)
        load_and_fire(1)

        def body(c, carry):
            reduce_and_out(c)
            @pl.when(c + 2 < NCP)
            def _():
                load_and_fire(c + 2)
            return carry
        lax.fori_loop(0, NCP, body, 0)

    pool(0, likedT)
    pool(1, dislikedT)
    pool(2, allergyT)
    pool(3, tagsT)

    pltpu.sync_copy(misc_v, outc.at[pl.ds(base, BC), pl.ds(0, 16)])


def _sc_plain_body(user_t, dish_t, store_t, cat_t,
                   uid, did, sid, cid,
                   age, utod, price, rating, itod, gid, udy, idy,
                   out2, out3,
                   misc_v, tmpf_v, tmpi_v, gidx_v, g64_v, g32_v, g16_v,
                   sem_p):
    base = _wid_base()
    f32 = jnp.float32
    i32 = jnp.int32

    def pack_scalar(col, src_hbm, is_int):
        tmp = tmpi_v if is_int else tmpf_v
        pltpu.sync_copy(src_hbm.at[pl.ds(base, BC)], tmp)
        colv = jnp.full((LANES,), col, i32)

        def body(g, carry):
            vals = tmp[pl.ds(g * LANES, LANES)]
            if is_int:
                vals = vals.astype(f32)
            rowv = lax.iota(i32, LANES) + g * LANES
            plsc.store_scatter(misc_v, [rowv, colv], vals)
            return carry
        lax.fori_loop(0, NG, body, 0)

    for col, (src, is_int) in enumerate([
            (age, False), (utod, False), (price, False), (rating, False),
            (itod, False), (gid, True), (udy, True), (idy, True)]):
        pack_scalar(col, src, is_int)

    def plain(table, ids_hbm, out_hbm, col0, width, buf):
        def stage(c):
            p = lax.rem(c, 2)
            off = pl.multiple_of(base + c * CG, CG)
            pltpu.sync_copy(ids_hbm.at[pl.ds(off, CG)], gidx_v.at[p])
            pltpu.make_async_copy(table.at[gidx_v.at[p]], buf.at[p],
                                  sem_p).start()

        def drain(c):
            p = lax.rem(c, 2)
            off = pl.multiple_of(base + c * CG, CG)
            pltpu.make_async_copy(table.at[gidx_v.at[p]], buf.at[p],
                                  sem_p).wait()
            pltpu.sync_copy(buf.at[p],
                            out_hbm.at[pl.ds(off, CG), pl.ds(col0, width)])

        stage(0)

        def body(c, carry):
            @pl.when(c + 1 < NCG)
            def _():
                stage(c + 1)
            drain(c)
            return carry
        lax.fori_loop(0, NCG, body, 0)

    plain(user_t, uid, out2, 0, 64, g64_v)
    plain(dish_t, did, out2, 64, 64, g64_v)
    plain(store_t, sid, out3, 0, 32, g32_v)
    plain(cat_t, cid, out3, 32, 16, g16_v)

    pltpu.sync_copy(misc_v, out3.at[pl.ds(base, BC), pl.ds(48, 16)])


@functools.cache
def _sc_pools_call():
    return pl.kernel(
        _sc_pools_body,
        out_type=(
            jax.ShapeDtypeStruct((B, 128), jnp.float32),
            jax.ShapeDtypeStruct((B, 128), jnp.float32),
        ),
        mesh=plsc.VectorSubcoreMesh(**_MESH),
        scratch_types=[
            pltpu.VMEM((2, T, CP), jnp.int32),
            pltpu.VMEM((2, T, CP, 32), jnp.bfloat16),
            pltpu.VMEM((CP, 32), jnp.float32),
            pltpu.VMEM((BC, 16), jnp.float32),
            pltpu.SemaphoreType.DMA,
        ],
        compiler_params=pltpu.CompilerParams(**_PARAMS),
    )


@functools.cache
def _sc_plain_call():
    return pl.kernel(
        _sc_plain_body,
        out_type=(
            jax.ShapeDtypeStruct((B, 128), jnp.float32),
            jax.ShapeDtypeStruct((B, 128), jnp.float32),
        ),
        mesh=plsc.VectorSubcoreMesh(**_MESH),
        scratch_types=[
            pltpu.VMEM((BC, 16), jnp.float32),
            pltpu.VMEM((BC,), jnp.float32),
            pltpu.VMEM((BC,), jnp.int32),
            pltpu.VMEM((2, CG), jnp.int32),
            pltpu.VMEM((2, CG, 64), jnp.float32),
            pltpu.VMEM((2, CG, 32), jnp.float32),
            pltpu.VMEM((2, CG, 16), jnp.float32),
            pltpu.SemaphoreType.DMA,
        ],
        compiler_params=pltpu.CompilerParams(**_PARAMS),
    )


BN = 4096


def _tc_body(o1, oc, o2, o3,
             Wu, bu, Wi, bi, gemb, uday, iday,
             ageW, ageb, utW, utb, prW, prb, rtW, rtb, itW, itb, row0,
             uo, io, doto):
    f32 = jnp.float32
    dot = functools.partial(lax.dot, preferred_element_type=f32)
    r0 = row0[...]
    o1_ = o1[...]
    oc_ = oc[...]
    o2_ = o2[...]
    o3_ = o3[...]

    def pool(acc, cnt):
        return (acc - (float(T) - cnt) * r0) / (cnt + 1e-8)

    liked = pool(o1_[:, 0:32], oc_[:, 0:1])
    disl = pool(o1_[:, 32:64], oc_[:, 1:2])
    alle = pool(o1_[:, 64:96], oc_[:, 2:3])
    tagv = pool(o1_[:, 96:128], oc_[:, 3:4])
    age = o3_[:, 48:49]
    utod = o3_[:, 49:50]
    price = o3_[:, 50:51]
    rating = o3_[:, 51:52]
    itod = o3_[:, 52:53]

    def onehot(col, n):
        ci = col.astype(jnp.int32)
        return (lax.broadcasted_iota(jnp.int32, (BN, n), 1) == ci).astype(f32)

    Wu_ = Wu[...]
    u = dot(o2_[:, 0:64], Wu_[0:64])
    u += dot(liked, Wu_[112:144])
    u += dot(disl, Wu_[144:176])
    u += dot(alle, Wu_[176:208])
    u += age * dot(ageW[...], Wu_[64:80])
    u += utod * dot(utW[...], Wu_[96:104])
    u += dot(onehot(o3_[:, 53:54], 3), dot(gemb[...], Wu_[80:96]))
    u += dot(onehot(o3_[:, 54:55], 7), dot(uday[...], Wu_[104:112]))
    u += bu[...] + dot(ageb[...], Wu_[64:80]) + dot(utb[...], Wu_[96:104])
    nu = jnp.sqrt(jnp.sum(u * u, axis=1, keepdims=True))
    un = u / jnp.maximum(nu, 1e-12)

    Wi_ = Wi[...]
    iv = dot(o2_[:, 64:128], Wi_[0:64])
    iv += dot(o3_[:, 0:32], Wi_[64:96])
    iv += dot(tagv, Wi_[96:128])
    iv += dot(o3_[:, 32:48], Wi_[128:144])
    iv += price * dot(prW[...], Wi_[144:160])
    iv += rating * dot(rtW[...], Wi_[160:168])
    iv += itod * dot(itW[...], Wi_[168:176])
    iv += dot(onehot(o3_[:, 55:56], 7), dot(iday[...], Wi_[176:184]))
    iv += (bi[...] + dot(prb[...], Wi_[144:160]) + dot(rtb[...], Wi_[160:168])
           + dot(itb[...], Wi_[168:176]))
    ni = jnp.sqrt(jnp.sum(iv * iv, axis=1, keepdims=True))
    ivn = iv / jnp.maximum(ni, 1e-12)

    uo[...] = un
    io[...] = ivn
    doto[...] = jnp.sum(un * ivn, axis=1, keepdims=True)


def _row_spec(k):
    return pl.BlockSpec((BN, k), lambda i: (i, 0))


def _full_spec(shape):
    return pl.BlockSpec(shape, lambda i: (0,) * len(shape))


def kernel(user_user_id, user_age, user_gender, user_time_of_day,
           user_day_of_week, user_liked_tags, user_disliked_tags,
           user_allergy_tags, item_dish_id, item_store_id, item_category,
           item_tags, item_price, item_rating, item_time_of_day,
           item_day_of_week, user_embedding, user_age_W, user_age_b,
           user_gender_emb, user_time_W, user_time_b, user_day_emb,
           dish_embedding, store_embedding, category_embedding,
           dish_price_W, dish_price_b, dish_rating_W, dish_rating_b,
           dish_time_W, dish_time_b, dish_day_emb, tag_embedding,
           user_proj_W, user_proj_b, item_proj_W, item_proj_b):
    i32 = jnp.int32
    f32 = jnp.float32

    o1, oc = _sc_pools_call()(
        tag_embedding.astype(jnp.bfloat16),
        user_liked_tags.astype(i32).T, user_disliked_tags.astype(i32).T,
        user_allergy_tags.astype(i32).T, item_tags.astype(i32).T)

    o2, o3 = _sc_plain_call()(
        user_embedding, dish_embedding, store_embedding, category_embedding,
        user_user_id.astype(i32), item_dish_id.astype(i32),
        item_store_id.astype(i32), item_category.astype(i32),
        user_age.astype(f32), user_time_of_day.astype(f32),
        item_price.astype(f32), item_rating.astype(f32),
        item_time_of_day.astype(f32), user_gender.astype(i32),
        user_day_of_week.astype(i32), item_day_of_week.astype(i32))

    perm = jnp.array(PERM32)
    row0 = tag_embedding[0:1, perm].astype(jnp.bfloat16).astype(jnp.float32)
    idxu = list(range(208))
    for s in (112, 144, 176):
        idxu[s:s + 32] = [s + p for p in PERM32]
    idxi = list(range(184))
    idxi[96:128] = [96 + p for p in PERM32]
    Wu_p = user_proj_W[jnp.array(idxu), :]
    Wi_p = item_proj_W[jnp.array(idxi), :]
    u_in, i_in = 208, 184
    weights = dict(
        Wu=(Wu_p, (u_in, D)), bu=(user_proj_b.reshape(1, D), (1, D)),
        Wi=(Wi_p, (i_in, D)), bi=(item_proj_b.reshape(1, D), (1, D)),
        gemb=(user_gender_emb, (3, 16)), uday=(user_day_emb, (7, 8)),
        iday=(dish_day_emb, (7, 8)),
        ageW=(user_age_W, (1, 16)), ageb=(user_age_b.reshape(1, 16), (1, 16)),
        utW=(user_time_W, (1, 8)), utb=(user_time_b.reshape(1, 8), (1, 8)),
        prW=(dish_price_W, (1, 16)), prb=(dish_price_b.reshape(1, 16), (1, 16)),
        rtW=(dish_rating_W, (1, 8)), rtb=(dish_rating_b.reshape(1, 8), (1, 8)),
        itW=(dish_time_W, (1, 8)), itb=(dish_time_b.reshape(1, 8), (1, 8)),
        row0=(row0, (1, 32)),
    )

    in_specs = ([_row_spec(128)] * 4
                + [_full_spec(s) for (_, s) in weights.values()])

    un, ivn, dotv = pl.pallas_call(
        _tc_body,
        grid=(B // BN,),
        in_specs=in_specs,
        out_specs=[_row_spec(D), _row_spec(D), _row_spec(1)],
        out_shape=[
            jax.ShapeDtypeStruct((B, D), f32),
            jax.ShapeDtypeStruct((B, D), f32),
            jax.ShapeDtypeStruct((B, 1), f32),
        ],
    )(o1, oc, o2, o3, *[w for (w, _) in weights.values()])

    return un, ivn, dotv.reshape(B)
